# baseline (speedup 1.0000x reference)
import functools

import jax
import jax.numpy as jnp
from jax import lax
from jax.experimental import pallas as pl
from jax.experimental.pallas import tpu as pltpu
from jax.experimental.pallas import tpu_sc as plsc

_NC = 2
_NS = 16
_NW = _NC * _NS
_CHUNK = 128


def _sc_gather_body(n_chunks, x_hbm, idx_hbm, out_hbm, idx_v, rows_v, sem):
    wid = lax.axis_index("s") * _NC + lax.axis_index("c")
    pltpu.sync_copy(idx_hbm.at[wid], idx_v)

    def issue(j, carry):
        pltpu.async_copy(
            x_hbm.at[idx_v.at[j]], rows_v.at[pl.ds(j * _CHUNK, _CHUNK)], sem)
        return carry

    lax.fori_loop(0, n_chunks, issue, 0, unroll=False)
    pltpu.make_async_copy(
        x_hbm.at[pl.ds(0, n_chunks * _CHUNK)], rows_v, sem).wait()
    pltpu.sync_copy(rows_v, out_hbm.at[wid])


def _sc_gather(x_flat, idx3):
    n_chunks = idx3.shape[1]
    mesh = plsc.VectorSubcoreMesh(
        core_axis_name="c", subcore_axis_name="s", num_cores=_NC,
        num_subcores=_NS)
    kern = pl.kernel(
        functools.partial(_sc_gather_body, n_chunks),
        out_type=jax.ShapeDtypeStruct((_NW, n_chunks * _CHUNK), jnp.uint32),
        mesh=mesh,
        scratch_types=[
            pltpu.VMEM((n_chunks, _CHUNK), jnp.int32),
            pltpu.VMEM((n_chunks * _CHUNK,), jnp.uint32),
            pltpu.SemaphoreType.DMA,
        ],
    )
    return kern(x_flat, idx3)


def _dense_body(bb, v, vq, x_ref, o_ref, flat_ref, xh_ref, xq_ref, sem):
    i = pl.program_id(0)
    x = x_ref[...]
    ul = lax.bitcast_convert_type(x[:, :vq], jnp.uint32)
    uh = lax.bitcast_convert_type(x[:, vq:], jnp.uint32)
    xh_ref[:, :v - vq] = uh
    xq_ref[...] = (ul >> 16) | (xh_ref[...] & jnp.uint32(0xFFFF0000))
    copies = [
        pltpu.make_async_copy(
            xq_ref.at[r], flat_ref.at[pl.ds((i * bb + r) * vq, vq)], sem)
        for r in range(bb)
    ]
    for c in copies:
        c.start()

    dense = jnp.sum(jnp.maximum(x, 0.0) + jnp.log(1.0 + jnp.exp(-jnp.abs(x))))

    @pl.when(i == 0)
    def _():
        o_ref[...] = jnp.zeros_like(o_ref)

    o_ref[...] += jnp.full((1, 1), dense, jnp.float32)
    for c in copies:
        c.wait()


def _corr_body(l_real, vq, ids_ref, m_ref, lo_ref, hi_ref, o_ref):
    ids = ids_ref[...]
    m = m_ref[...]
    vals = jnp.where(ids >= vq, hi_ref[...], lo_ref[...])
    lu = (l_real + 7) // 8 * 8
    eq = ids[:, :lu, None] == ids[:, None, :]
    mt = jnp.sum(jnp.where(eq, m[:, :lu, None], 0.0), axis=1)
    w = jnp.where(mt != 0.0, m * jnp.minimum(mt, 1.0) / mt, 0.0)
    corr = jnp.sum(w * vals)

    @pl.when(pl.program_id(0) == 0)
    def _():
        o_ref[...] = jnp.zeros_like(o_ref)

    o_ref[...] += jnp.full((1, 1), corr, jnp.float32)


def kernel(sparse_repr, input_ids, attention_mask):
    b, v = sparse_repr.shape
    l = input_ids.shape[1]
    ids = input_ids.astype(jnp.int32)
    mask = attention_mask.astype(jnp.float32)

    bb = 8
    assert v % 2 == 0
    vq = (v // 2 + 127) // 128 * 128
    dense_tot, x_flat = pl.pallas_call(
        functools.partial(_dense_body, bb, v, vq),
        grid=(b // bb,),
        in_specs=[pl.BlockSpec((bb, v), lambda i: (i, 0))],
        out_specs=[
            pl.BlockSpec((1, 1), lambda i: (0, 0)),
            pl.BlockSpec(memory_space=pl.ANY),
        ],
        out_shape=[
            jax.ShapeDtypeStruct((1, 1), jnp.float32),
            jax.ShapeDtypeStruct((b * vq,), jnp.uint32),
        ],
        scratch_shapes=[
            pltpu.VMEM((bb, vq), jnp.uint32),
            pltpu.VMEM((bb, vq), jnp.uint32),
            pltpu.SemaphoreType.DMA,
        ],
    )(sparse_repr)

    col = jnp.where(ids >= vq, ids - vq, ids)
    flat_idx = (col + jnp.arange(b, dtype=jnp.int32)[:, None] * vq).reshape(-1)
    assert (b * l) % (_NW * _CHUNK) == 0
    idx3 = flat_idx.reshape(_NW, -1, _CHUNK)
    pair = lax.bitcast_convert_type(
        _sc_gather(x_flat, idx3).reshape(b, l), jnp.bfloat16)
    lo = pair[:, :, 0].astype(jnp.float32)
    hi = pair[:, :, 1].astype(jnp.float32)

    lp = (l + 127) // 128 * 128
    ids_p = jnp.pad(ids, ((0, 0), (0, lp - l)), constant_values=-1)
    m_p = jnp.pad(mask, ((0, 0), (0, lp - l)))
    lo_p = jnp.pad(lo, ((0, 0), (0, lp - l)))
    hi_p = jnp.pad(hi, ((0, 0), (0, lp - l)))

    bc = 16
    corr_tot = pl.pallas_call(
        functools.partial(_corr_body, l, vq),
        grid=(b // bc,),
        in_specs=[
            pl.BlockSpec((bc, lp), lambda i: (i, 0)),
            pl.BlockSpec((bc, lp), lambda i: (i, 0)),
            pl.BlockSpec((bc, lp), lambda i: (i, 0)),
            pl.BlockSpec((bc, lp), lambda i: (i, 0)),
        ],
        out_specs=pl.BlockSpec((1, 1), lambda i: (0, 0)),
        out_shape=jax.ShapeDtypeStruct((1, 1), jnp.float32),
    )(ids_p, m_p, lo_p, hi_p)

    return (dense_tot[0, 0] - corr_tot[0, 0]) / (b * v)


# speedup vs baseline: 1.0632x; 1.0632x over previous
import functools

import jax
import jax.numpy as jnp
from jax import lax
from jax.experimental import pallas as pl
from jax.experimental.pallas import tpu as pltpu
from jax.experimental.pallas import tpu_sc as plsc

_NC = 2
_NS = 16
_NW = _NC * _NS
_CHUNK = 128


def _sc_gather_body(n_chunks, x_hbm, idx_hbm, out_hbm, idx_v, rows_v, sem):
    wid = lax.axis_index("s") * _NC + lax.axis_index("c")
    pltpu.sync_copy(idx_hbm.at[wid], idx_v)

    def issue(j, carry):
        pltpu.async_copy(
            x_hbm.at[idx_v.at[j]], rows_v.at[pl.ds(j * _CHUNK, _CHUNK)], sem)
        return carry

    lax.fori_loop(0, n_chunks, issue, 0, unroll=False)
    pltpu.make_async_copy(
        x_hbm.at[pl.ds(0, n_chunks * _CHUNK)], rows_v, sem).wait()
    pltpu.sync_copy(rows_v, out_hbm.at[wid])


def _sc_gather(x_flat, idx3):
    n_chunks = idx3.shape[1]
    mesh = plsc.VectorSubcoreMesh(
        core_axis_name="c", subcore_axis_name="s", num_cores=_NC,
        num_subcores=_NS)
    kern = pl.kernel(
        functools.partial(_sc_gather_body, n_chunks),
        out_type=jax.ShapeDtypeStruct((_NW, n_chunks * _CHUNK), jnp.float32),
        mesh=mesh,
        scratch_types=[
            pltpu.VMEM((n_chunks, _CHUNK), jnp.int32),
            pltpu.VMEM((n_chunks * _CHUNK,), jnp.float32),
            pltpu.SemaphoreType.DMA,
        ],
    )
    return kern(x_flat, idx3)


def _dense_body(bb, v, vp, x_ref, o_ref, flat_ref, xp_ref, sem):
    i = pl.program_id(0)
    x = x_ref[...]
    xp_ref[:, :v] = x
    copies = [
        pltpu.make_async_copy(
            xp_ref.at[r], flat_ref.at[pl.ds((i * bb + r) * vp, vp)], sem)
        for r in range(bb)
    ]
    for c in copies:
        c.start()

    dense = jnp.sum(jnp.maximum(x, 0.0) + jnp.log(1.0 + jnp.exp(-jnp.abs(x))))

    @pl.when(i == 0)
    def _():
        o_ref[...] = jnp.zeros_like(o_ref)

    o_ref[...] += jnp.full((1, 1), dense, jnp.float32)
    for c in copies:
        c.wait()


def _corr_body(l_real, ids_ref, m_ref, vals_ref, o_ref):
    ids = ids_ref[...]
    m = m_ref[...]
    vals = vals_ref[...]
    lu = (l_real + 7) // 8 * 8
    eq = ids[:, :lu, None] == ids[:, None, :]
    mt = jnp.sum(jnp.where(eq, m[:, :lu, None], 0.0), axis=1)
    w = jnp.where(mt != 0.0, m * jnp.minimum(mt, 1.0) / mt, 0.0)
    corr = jnp.sum(w * vals)

    @pl.when(pl.program_id(0) == 0)
    def _():
        o_ref[...] = jnp.zeros_like(o_ref)

    o_ref[...] += jnp.full((1, 1), corr, jnp.float32)


def kernel(sparse_repr, input_ids, attention_mask):
    b, v = sparse_repr.shape
    l = input_ids.shape[1]
    ids = input_ids.astype(jnp.int32)
    mask = attention_mask.astype(jnp.float32)

    bb = 8
    vp = (v + 127) // 128 * 128
    hb = b // 2

    def dense_half(row0):
        return pl.pallas_call(
            functools.partial(_dense_body, bb, v, vp),
            grid=(hb // bb,),
            in_specs=[
                pl.BlockSpec((bb, v), lambda i, r0=row0: (i + r0 // bb, 0))],
            out_specs=[
                pl.BlockSpec((1, 1), lambda i: (0, 0)),
                pl.BlockSpec(memory_space=pl.ANY),
            ],
            out_shape=[
                jax.ShapeDtypeStruct((1, 1), jnp.float32),
                jax.ShapeDtypeStruct((hb * vp,), jnp.float32),
            ],
            scratch_shapes=[
                pltpu.VMEM((bb, vp), jnp.float32),
                pltpu.SemaphoreType.DMA,
            ],
        )(sparse_repr)

    d0, flat0 = dense_half(0)
    d1, flat1 = dense_half(hb)
    dense_tot = d0 + d1

    assert (hb * l) % (_NW * _CHUNK) == 0
    rows_loc = jnp.arange(hb, dtype=jnp.int32)[:, None] * vp

    def gather_half(flat_h, ids_h):
        idx3 = (ids_h + rows_loc).reshape(_NW, -1, _CHUNK)
        return _sc_gather(flat_h, idx3).reshape(hb, l)

    vals = jnp.concatenate(
        [gather_half(flat0, ids[:hb]), gather_half(flat1, ids[hb:])], axis=0)

    lp = (l + 127) // 128 * 128
    ids_p = jnp.pad(ids, ((0, 0), (0, lp - l)), constant_values=-1)
    m_p = jnp.pad(mask, ((0, 0), (0, lp - l)))
    vals_p = jnp.pad(vals, ((0, 0), (0, lp - l)))

    bc = 16
    corr_tot = pl.pallas_call(
        functools.partial(_corr_body, l),
        grid=(b // bc,),
        in_specs=[
            pl.BlockSpec((bc, lp), lambda i: (i, 0)),
            pl.BlockSpec((bc, lp), lambda i: (i, 0)),
            pl.BlockSpec((bc, lp), lambda i: (i, 0)),
        ],
        out_specs=pl.BlockSpec((1, 1), lambda i: (0, 0)),
        out_shape=jax.ShapeDtypeStruct((1, 1), jnp.float32),
    )(ids_p, m_p, vals_p)

    return (dense_tot[0, 0] - corr_tot[0, 0]) / (b * v)
